# baseline (speedup 1.0000x reference)
import jax
import jax.numpy as jnp
from jax import lax
from jax.experimental import pallas as pl
from jax.experimental.pallas import tpu as pltpu
from jax.experimental.pallas import tpu_sc as plsc

N = 50000
D = 64
DH = 32
NC, NS = 2, 16
CH = 128
BB = 14
NB = 28
NCH = NB * BB
NCHP = 424
E = 800000
EPAD = NS * NCH * CH
NPAD = 51200
RPT = NPAD // NS
TRASH = N
RING = 4
V4 = NPAD * DH // 128

_SC_PARAMS = pltpu.CompilerParams(use_tc_tiling_on_sc=False)


def _drain(acc_sh, dst, c, s):
    pltpu.sync_copy(acc_sh.at[pl.ds(s * RPT, RPT)],
                    dst.at[c, pl.ds(s * RPT, RPT)])


def _deg_scatter(idxarr, s, si, ones_v, acc_sh, sem_set):
    pltpu.async_copy(idxarr.at[s, pl.ds(0, BB)], si[0], sem_set[0])
    pltpu.async_copy(idxarr.at[s, pl.ds(BB, BB)], si[1], sem_set[1])

    def pair(p, carry):
        for k in range(2):
            pltpu.make_async_copy(idxarr.at[s, pl.ds(0, BB)], si[k],
                                  sem_set[k]).wait()

            def step(j, cc):
                pltpu.sync_copy(ones_v, acc_sh.at[si[k].at[j]], add=True)
                return cc

            lax.fori_loop(0, BB, step, 0)
            pltpu.async_copy(idxarr.at[s, pl.ds((2 * p + 2 + k) * BB, BB)],
                             si[k], sem_set[k])
        return carry

    lax.fori_loop(0, NB // 2, pair, 0)
    for k in range(2):
        pltpu.make_async_copy(idxarr.at[s, pl.ds(0, BB)], si[k],
                              sem_set[k]).wait()


def _msg_pass(tbl, gidx, sidx, zeros_h, acc_out, c, s,
              gi, si, bufs, acc_sh, sem_g, sem_set):
    pltpu.sync_copy(zeros_h, acc_sh.at[pl.ds(s * RPT, RPT)])
    plsc.subcore_barrier()

    def prime_set(k, blk):
        pltpu.async_copy(gidx.at[c, s, pl.ds(blk * BB, BB)], gi[k],
                         sem_set[k])
        pltpu.async_copy(sidx.at[s, pl.ds(blk * BB, BB)], si[k], sem_set[k])

    def wait_set(k):
        pltpu.make_async_copy(gidx.at[c, s, pl.ds(0, BB)], gi[k],
                              sem_set[k]).wait()
        pltpu.make_async_copy(sidx.at[s, pl.ds(0, BB)], si[k],
                              sem_set[k]).wait()

    prime_set(0, 0)
    prime_set(1, 1)

    def idx_rows(k):
        if k < BB:
            return gi[0].at[k], si[0].at[k]
        return gi[1].at[k - BB], si[1].at[k - BB]

    def pair(p, carry):
        wait_set(0)
        descs = {}
        for r in range(RING):
            g, _ = idx_rows(r)
            descs[r] = pltpu.async_copy(tbl.at[g], bufs[r], sem_g[r])
        for j in range(2 * BB):
            m = j % RING
            descs[m].wait()
            _, srow = idx_rows(j)
            pltpu.sync_copy(bufs[m], acc_sh.at[srow], add=True)
            nj = j + RING
            if nj == BB:
                wait_set(1)
            if nj < 2 * BB:
                g, _ = idx_rows(nj)
                descs[m] = pltpu.async_copy(tbl.at[g], bufs[m], sem_g[m])
            if j == BB - 1:
                prime_set(0, 2 * p + 2)
            if j == 2 * BB - 1:
                prime_set(1, 2 * p + 3)
        return carry

    lax.fori_loop(0, NB // 2, pair, 0)
    wait_set(0)
    wait_set(1)
    plsc.subcore_barrier()
    _drain(acc_sh, acc_out, c, s)


def _fwd_body(*refs):
    (tbl, gidx, ed, es, ones_h, zeros_h, deg_out, acc_out,
     gi0, gi1, si0, si1, b0, b1, b2, b3,
     acc_sh, sr0, sr1, sr2, sr3, ss0, ss1) = refs
    c = lax.axis_index("c")
    s = lax.axis_index("s")
    gi, si = [gi0, gi1], [si0, si1]
    bufs = [b0, b1, b2, b3]
    sem_g, sem_set = [sr0, sr1, sr2, sr3], [ss0, ss1]

    pltpu.sync_copy(ones_h, b0)
    pltpu.sync_copy(zeros_h, acc_sh.at[pl.ds(s * RPT, RPT)])
    plsc.subcore_barrier()

    @pl.when(c == 0)
    def _():
        _deg_scatter(ed, s, si, b0, acc_sh, sem_set)

    @pl.when(c == 1)
    def _():
        _deg_scatter(es, s, si, b0, acc_sh, sem_set)

    plsc.subcore_barrier()
    _drain(acc_sh, deg_out, c, s)

    _msg_pass(tbl, gidx, ed, zeros_h, acc_out,
              c, s, gi, si, bufs, acc_sh, sem_g, sem_set)


def _bwd_body(*refs):
    (tbl, gidx, es, zeros_h, acc_out,
     gi0, gi1, si0, si1, b0, b1, b2, b3,
     acc_sh, sr0, sr1, sr2, sr3, ss0, ss1) = refs
    c = lax.axis_index("c")
    s = lax.axis_index("s")
    _msg_pass(tbl, gidx, es, zeros_h, acc_out,
              c, s, [gi0, gi1], [si0, si1], [b0, b1, b2, b3],
              acc_sh, [sr0, sr1, sr2, sr3], [ss0, ss1])


_SC_SCRATCH = [
    pltpu.VMEM((BB, CH), jnp.int32),
    pltpu.VMEM((BB, CH), jnp.int32),
    pltpu.VMEM((BB, CH), jnp.int32),
    pltpu.VMEM((BB, CH), jnp.int32),
    pltpu.VMEM((CH, DH), jnp.float32),
    pltpu.VMEM((CH, DH), jnp.float32),
    pltpu.VMEM((CH, DH), jnp.float32),
    pltpu.VMEM((CH, DH), jnp.float32),
    pltpu.VMEM_SHARED((NPAD, DH), jnp.float32),
    pltpu.SemaphoreType.DMA,
    pltpu.SemaphoreType.DMA,
    pltpu.SemaphoreType.DMA,
    pltpu.SemaphoreType.DMA,
    pltpu.SemaphoreType.DMA,
    pltpu.SemaphoreType.DMA,
]

_MESH = plsc.VectorSubcoreMesh(core_axis_name="c", subcore_axis_name="s")

_fwd_kernel = pl.kernel(
    _fwd_body,
    out_type=(jax.ShapeDtypeStruct((NC, NPAD, DH), jnp.float32),
              jax.ShapeDtypeStruct((NC, NPAD, DH), jnp.float32)),
    mesh=_MESH,
    compiler_params=_SC_PARAMS,
    scratch_types=_SC_SCRATCH,
)

_bwd_kernel = pl.kernel(
    _bwd_body,
    out_type=jax.ShapeDtypeStruct((NC, NPAD, DH), jnp.float32),
    mesh=_MESH,
    compiler_params=_SC_PARAMS,
    scratch_types=_SC_SCRATCH,
)


_BRN = 512


def _norm_body(x_ref, d_ref, o_ref):
    o_ref[...] = x_ref[...] * (1.0 / jnp.maximum(d_ref[...], 1.0))


_norm = pl.pallas_call(
    _norm_body,
    out_shape=jax.ShapeDtypeStruct((NC, V4, 128), jnp.float32),
    grid=(V4 // _BRN,),
    in_specs=[
        pl.BlockSpec((NC, _BRN, 128), lambda j: (0, j, 0)),
        pl.BlockSpec((_BRN, 128), lambda j: (j, 0)),
    ],
    out_specs=pl.BlockSpec((NC, _BRN, 128), lambda j: (0, j, 0)),
)


def _tile_idx(e, padval):
    pad = jnp.full((EPAD - E,), padval, jnp.int32)
    t = jnp.concatenate([e, pad]).reshape(NS, NCH, CH)
    return jnp.pad(t, ((0, 0), (0, NCHP - NCH), (0, 0)),
                   constant_values=padval)


def kernel(feat_src, feat_dst, edge_index):
    e_src = edge_index[0].astype(jnp.int32)
    e_dst = edge_index[1].astype(jnp.int32)
    es_t = _tile_idx(e_src, TRASH)
    ed_t = _tile_idx(e_dst, TRASH)
    es_0 = _tile_idx(e_src, 0)
    ed_0 = _tile_idx(e_dst, 0)
    gidx_f = jnp.stack([2 * es_0, 2 * es_0 + 1])
    gidx_b = jnp.stack([ed_0, ed_0 + NPAD])

    ones_h = jnp.ones((CH, DH), jnp.float32)
    zeros_h = jnp.zeros((RPT, DH), jnp.float32)

    tbl_f = feat_src.reshape(2 * N, DH)

    degs, acc_f = _fwd_kernel(tbl_f, gidx_f, ed_t, es_t, ones_h, zeros_h)

    rstn = _norm(acc_f.reshape(NC, V4, 128), degs[0].reshape(V4, 128))
    rstn_v = rstn.reshape(NC, NPAD, DH)
    rst = jnp.concatenate([rstn_v[0, :N], rstn_v[1, :N]], axis=1)

    acc_b = _bwd_kernel(rstn.reshape(NC * NPAD, DH), gidx_b, es_t, zeros_h)
    bn = _norm(acc_b.reshape(NC, V4, 128), degs[1].reshape(V4, 128))
    bn_v = bn.reshape(NC, NPAD, DH)
    bsrc = jnp.concatenate([bn_v[0, :N], bn_v[1, :N]], axis=1)
    return (bsrc, rst)


# speedup vs baseline: 1.0482x; 1.0482x over previous
import jax
import jax.numpy as jnp
from jax import lax
from jax.experimental import pallas as pl
from jax.experimental.pallas import tpu as pltpu
from jax.experimental.pallas import tpu_sc as plsc

N = 50000
D = 64
DH = 32
NC, NS = 2, 16
CH = 128
BB = 14
NB = 28
NCH = NB * BB
NCHP = 424
E = 800000
EPAD = NS * NCH * CH
NPAD = 51200
RPT = NPAD // NS
TRASH = N
RING = 4
V4 = NPAD * DH // 128

_SC_PARAMS = pltpu.CompilerParams(use_tc_tiling_on_sc=False)


def _drain(acc_sh, dst, c, s):
    pltpu.sync_copy(acc_sh.at[pl.ds(s * RPT, RPT)],
                    dst.at[c, pl.ds(s * RPT, RPT)])


def _deg_scatter(idxarr, s, si, ones_v, acc_sh, sem_set):
    pltpu.async_copy(idxarr.at[s, pl.ds(0, BB)], si[0], sem_set[0])
    pltpu.async_copy(idxarr.at[s, pl.ds(BB, BB)], si[1], sem_set[1])

    def pair(p, carry):
        for k in range(2):
            pltpu.make_async_copy(idxarr.at[s, pl.ds(0, BB)], si[k],
                                  sem_set[k]).wait()

            def step(j, cc):
                pltpu.sync_copy(ones_v, acc_sh.at[si[k].at[j]], add=True)
                return cc

            lax.fori_loop(0, BB, step, 0)
            pltpu.async_copy(idxarr.at[s, pl.ds((2 * p + 2 + k) * BB, BB)],
                             si[k], sem_set[k])
        return carry

    lax.fori_loop(0, NB // 2, pair, 0)
    for k in range(2):
        pltpu.make_async_copy(idxarr.at[s, pl.ds(0, BB)], si[k],
                              sem_set[k]).wait()


def _msg_pass(tbl, gidx, sidx, zeros_h, acc_out, c, s,
              gi, si, bufs, acc_sh, sem_g, sem_set):
    pltpu.sync_copy(zeros_h, acc_sh.at[pl.ds(s * RPT, RPT)])
    plsc.subcore_barrier()

    def prime_set(k, blk):
        pltpu.async_copy(gidx.at[c, s, pl.ds(blk * BB, BB)], gi[k],
                         sem_set[k])
        pltpu.async_copy(sidx.at[s, pl.ds(blk * BB, BB)], si[k], sem_set[k])

    def wait_set(k):
        pltpu.make_async_copy(gidx.at[c, s, pl.ds(0, BB)], gi[k],
                              sem_set[k]).wait()
        pltpu.make_async_copy(sidx.at[s, pl.ds(0, BB)], si[k],
                              sem_set[k]).wait()

    prime_set(0, 0)
    prime_set(1, 1)

    def idx_rows(k):
        if k < BB:
            return gi[0].at[k], si[0].at[k]
        return gi[1].at[k - BB], si[1].at[k - BB]

    def pair(p, carry):
        wait_set(0)
        descs = {}
        for r in range(RING):
            g, _ = idx_rows(r)
            descs[r] = pltpu.async_copy(tbl.at[g], bufs[r], sem_g[r])
        for j in range(2 * BB):
            m = j % RING
            descs[m].wait()
            _, srow = idx_rows(j)
            pltpu.sync_copy(bufs[m], acc_sh.at[srow], add=True)
            nj = j + RING
            if nj == BB:
                wait_set(1)
            if nj < 2 * BB:
                g, _ = idx_rows(nj)
                descs[m] = pltpu.async_copy(tbl.at[g], bufs[m], sem_g[m])
            if j == BB - 1:
                prime_set(0, 2 * p + 2)
            if j == 2 * BB - 1:
                prime_set(1, 2 * p + 3)
        return carry

    lax.fori_loop(0, NB // 2, pair, 0)
    wait_set(0)
    wait_set(1)
    plsc.subcore_barrier()
    _drain(acc_sh, acc_out, c, s)


def _fwd_body(*refs):
    (tbl, gidx, ed, es, ones_h, zeros_h, deg_out, acc_out,
     gi0, gi1, si0, si1, b0, b1, b2, b3,
     acc_sh, sr0, sr1, sr2, sr3, ss0, ss1) = refs
    c = lax.axis_index("c")
    s = lax.axis_index("s")
    gi, si = [gi0, gi1], [si0, si1]
    bufs = [b0, b1, b2, b3]
    sem_g, sem_set = [sr0, sr1, sr2, sr3], [ss0, ss1]

    pltpu.sync_copy(ones_h, b0)
    pltpu.sync_copy(zeros_h, acc_sh.at[pl.ds(s * RPT, RPT)])
    plsc.subcore_barrier()

    @pl.when(c == 0)
    def _():
        _deg_scatter(ed, s, si, b0, acc_sh, sem_set)

    @pl.when(c == 1)
    def _():
        _deg_scatter(es, s, si, b0, acc_sh, sem_set)

    plsc.subcore_barrier()
    _drain(acc_sh, deg_out, c, s)

    _msg_pass(tbl, gidx, ed, zeros_h, acc_out,
              c, s, gi, si, bufs, acc_sh, sem_g, sem_set)


def _bwd_body(*refs):
    (tbl, gidx, es, zeros_h, acc_out,
     gi0, gi1, si0, si1, b0, b1, b2, b3,
     acc_sh, sr0, sr1, sr2, sr3, ss0, ss1) = refs
    c = lax.axis_index("c")
    s = lax.axis_index("s")
    _msg_pass(tbl, gidx, es, zeros_h, acc_out,
              c, s, [gi0, gi1], [si0, si1], [b0, b1, b2, b3],
              acc_sh, [sr0, sr1, sr2, sr3], [ss0, ss1])


_SC_SCRATCH = [
    pltpu.VMEM((BB, CH), jnp.int32),
    pltpu.VMEM((BB, CH), jnp.int32),
    pltpu.VMEM((BB, CH), jnp.int32),
    pltpu.VMEM((BB, CH), jnp.int32),
    pltpu.VMEM((CH, DH), jnp.float32),
    pltpu.VMEM((CH, DH), jnp.float32),
    pltpu.VMEM((CH, DH), jnp.float32),
    pltpu.VMEM((CH, DH), jnp.float32),
    pltpu.VMEM_SHARED((NPAD, DH), jnp.float32),
    pltpu.SemaphoreType.DMA,
    pltpu.SemaphoreType.DMA,
    pltpu.SemaphoreType.DMA,
    pltpu.SemaphoreType.DMA,
    pltpu.SemaphoreType.DMA,
    pltpu.SemaphoreType.DMA,
]

_MESH = plsc.VectorSubcoreMesh(core_axis_name="c", subcore_axis_name="s")

_fwd_kernel = pl.kernel(
    _fwd_body,
    out_type=(jax.ShapeDtypeStruct((NC, NPAD, DH), jnp.float32),
              jax.ShapeDtypeStruct((NC, NPAD, DH), jnp.float32)),
    mesh=_MESH,
    compiler_params=_SC_PARAMS,
    scratch_types=_SC_SCRATCH,
)

_bwd_kernel = pl.kernel(
    _bwd_body,
    out_type=jax.ShapeDtypeStruct((NC, NPAD, DH), jnp.float32),
    mesh=_MESH,
    compiler_params=_SC_PARAMS,
    scratch_types=_SC_SCRATCH,
)


_BRN = 512


def _norm_body(x_ref, d_ref, o_ref):
    o_ref[...] = x_ref[...] * (1.0 / jnp.maximum(d_ref[...], 1.0))


_norm = pl.pallas_call(
    _norm_body,
    out_shape=jax.ShapeDtypeStruct((NC, V4, 128), jnp.float32),
    grid=(V4 // _BRN,),
    in_specs=[
        pl.BlockSpec((NC, _BRN, 128), lambda j: (0, j, 0)),
        pl.BlockSpec((_BRN, 128), lambda j: (j, 0)),
    ],
    out_specs=pl.BlockSpec((NC, _BRN, 128), lambda j: (0, j, 0)),
)


def _tile_idx(e, padval):
    pad = jnp.full((EPAD - E,), padval, jnp.int32)
    t = jnp.concatenate([e, pad]).reshape(NS, NCH, CH)
    return jnp.pad(t, ((0, 0), (0, NCHP - NCH), (0, 0)),
                   constant_values=padval)


def kernel(feat_src, feat_dst, edge_index):
    e_src = edge_index[0].astype(jnp.int32)
    e_dst = edge_index[1].astype(jnp.int32)
    es_t = _tile_idx(e_src, TRASH)
    ed_t = _tile_idx(e_dst, TRASH)
    es_0 = _tile_idx(e_src, 0)
    ed_0 = _tile_idx(e_dst, 0)
    gidx_f = jnp.stack([2 * es_0, 2 * es_0 + 1])
    gidx_b = jnp.stack([ed_0, ed_0 + NPAD])

    ones_h = jnp.ones((CH, DH), jnp.float32)
    zeros_h = jnp.zeros((RPT, DH), jnp.float32)

    tbl_f = feat_src.reshape(2 * N, DH)

    degs, acc_f = _fwd_kernel(tbl_f, gidx_f, ed_t, es_t, ones_h, zeros_h)

    rstn = _norm(acc_f.reshape(NC, V4, 128), degs[0].reshape(V4, 128))
    rst = (rstn.reshape(NC, NPAD, DH)[:, :N]
           .transpose(1, 0, 2).reshape(N, D))

    acc_b = _bwd_kernel(rstn.reshape(NC * NPAD, DH), gidx_b, es_t, zeros_h)
    bn = _norm(acc_b.reshape(NC, V4, 128), degs[1].reshape(V4, 128))
    bsrc = (bn.reshape(NC, NPAD, DH)[:, :N]
            .transpose(1, 0, 2).reshape(N, D))
    return (bsrc, rst)
